# Initial kernel scaffold; baseline (speedup 1.0000x reference)
import jax
import jax.numpy as jnp
from jax.experimental import pallas as pl


def kernel(x, index, dim_size):
    raise NotImplementedError("write your pallas kernel here")


# speedup vs baseline: 4.5193x; 4.5193x over previous
import functools

import jax
import jax.numpy as jnp
from jax import lax
from jax.experimental import pallas as pl
from jax.experimental.pallas import tpu as pltpu
from jax.experimental.pallas import tpu_sc as plsc

N_NODES = 10000
N_EDGES = 320000
D = 128
NC, NS = 2, 16
NW = NC * NS
E_W = N_EDGES // NW
CHUNK = 80
IDX_ROWS = E_W // CHUNK
ROW_BASE = 624
ROW_SPAN = 640


@functools.partial(
    pl.kernel,
    out_type=jax.ShapeDtypeStruct((NC, N_NODES, D), jnp.float32),
    mesh=plsc.VectorSubcoreMesh(core_axis_name="c", subcore_axis_name="s"),
    scratch_types=[
        pltpu.VMEM((IDX_ROWS, CHUNK), jnp.int32),
        pltpu.VMEM((CHUNK, D), jnp.float32),
        pltpu.VMEM_SHARED((N_NODES, D), jnp.float32),
    ],
)
def _seg_sum_sc(x_hbm, idx_hbm, zeros_hbm, partial_hbm, idx_v, xbuf, acc):
    c = lax.axis_index("c")
    s = lax.axis_index("s")
    wid = s * NC + c

    pltpu.sync_copy(zeros_hbm, acc.at[pl.ds(s * ROW_BASE, ROW_SPAN)])
    pltpu.sync_copy(idx_hbm.at[wid], idx_v)
    plsc.subcore_barrier()

    ebase = wid * E_W

    def body(j, carry):
        pltpu.sync_copy(x_hbm.at[pl.ds(ebase + j * CHUNK, CHUNK)], xbuf)
        pltpu.sync_copy(xbuf, acc.at[idx_v.at[j]], add=True)
        return carry

    lax.fori_loop(0, IDX_ROWS, body, 0)

    plsc.subcore_barrier()
    pltpu.sync_copy(
        acc.at[pl.ds(s * ROW_BASE, ROW_SPAN)],
        partial_hbm.at[c, pl.ds(s * ROW_BASE, ROW_SPAN)],
    )


def _combine_body(p_ref, o_ref):
    o_ref[...] = p_ref[0] + p_ref[1]


_N_BLK = 10


def _combine(partial):
    return pl.pallas_call(
        _combine_body,
        grid=(_N_BLK,),
        in_specs=[
            pl.BlockSpec((NC, N_NODES // _N_BLK, D), lambda i: (0, i, 0))
        ],
        out_specs=pl.BlockSpec((N_NODES // _N_BLK, D), lambda i: (i, 0)),
        out_shape=jax.ShapeDtypeStruct((N_NODES, D), jnp.float32),
    )(partial)


def kernel(x, index, dim_size):
    idx2d = index.astype(jnp.int32).reshape(NW, IDX_ROWS, CHUNK)
    zeros = jnp.zeros((ROW_SPAN, D), jnp.float32)
    partial = _seg_sum_sc(x, idx2d, zeros)
    return _combine(partial)


# speedup vs baseline: 7.0801x; 1.5666x over previous
import functools

import jax
import jax.numpy as jnp
from jax import lax
from jax.experimental import pallas as pl
from jax.experimental.pallas import tpu as pltpu
from jax.experimental.pallas import tpu_sc as plsc

N_NODES = 10000
N_EDGES = 320000
D = 128
NC, NS = 2, 16
NW = NC * NS
E_W = N_EDGES // NW
CHUNK = 80
IDX_ROWS = E_W // CHUNK
ROW_BASE = 624
ROW_SPAN = 640


@functools.partial(
    pl.kernel,
    out_type=jax.ShapeDtypeStruct((NC, N_NODES, D), jnp.float32),
    mesh=plsc.VectorSubcoreMesh(core_axis_name="c", subcore_axis_name="s"),
    scratch_types=[
        pltpu.VMEM((IDX_ROWS, CHUNK), jnp.int32),
        pltpu.VMEM((CHUNK, D), jnp.float32),
        pltpu.VMEM((CHUNK, D), jnp.float32),
        pltpu.VMEM_SHARED((N_NODES, D), jnp.float32),
        pltpu.SemaphoreType.DMA,
        pltpu.SemaphoreType.DMA,
    ],
)
def _seg_sum_sc(x_hbm, idx_hbm, zeros_hbm, partial_hbm, idx_v, xbuf0, xbuf1,
                acc, sem0, sem1):
    c = lax.axis_index("c")
    s = lax.axis_index("s")
    wid = s * NC + c

    pltpu.sync_copy(zeros_hbm, acc.at[pl.ds(s * ROW_BASE, ROW_SPAN)])
    pltpu.sync_copy(idx_hbm.at[wid], idx_v)
    plsc.subcore_barrier()

    ebase = wid * E_W

    def gather(j, buf, sem):
        pltpu.async_copy(x_hbm.at[pl.ds(ebase + j * CHUNK, CHUNK)], buf, sem)

    def drain(buf, sem):
        pltpu.make_async_copy(x_hbm.at[pl.ds(ebase, CHUNK)], buf, sem).wait()

    gather(0, xbuf0, sem0)

    def body(p, carry):
        j = 2 * p
        gather(j + 1, xbuf1, sem1)
        drain(xbuf0, sem0)
        pltpu.sync_copy(xbuf0, acc.at[idx_v.at[j]], add=True)
        gather(j + 2, xbuf0, sem0)
        drain(xbuf1, sem1)
        pltpu.sync_copy(xbuf1, acc.at[idx_v.at[j + 1]], add=True)
        return carry

    lax.fori_loop(0, (IDX_ROWS - 1) // 2, body, 0)
    drain(xbuf0, sem0)
    pltpu.sync_copy(xbuf0, acc.at[idx_v.at[IDX_ROWS - 1]], add=True)

    plsc.subcore_barrier()
    pltpu.sync_copy(
        acc.at[pl.ds(s * ROW_BASE, ROW_SPAN)],
        partial_hbm.at[c, pl.ds(s * ROW_BASE, ROW_SPAN)],
    )


def _combine_body(p_ref, o_ref):
    o_ref[...] = p_ref[0] + p_ref[1]


_N_BLK = 10


def _combine(partial):
    return pl.pallas_call(
        _combine_body,
        grid=(_N_BLK,),
        in_specs=[
            pl.BlockSpec((NC, N_NODES // _N_BLK, D), lambda i: (0, i, 0))
        ],
        out_specs=pl.BlockSpec((N_NODES // _N_BLK, D), lambda i: (i, 0)),
        out_shape=jax.ShapeDtypeStruct((N_NODES, D), jnp.float32),
    )(partial)


def kernel(x, index, dim_size):
    idx2d = index.astype(jnp.int32).reshape(NW, IDX_ROWS, CHUNK)
    zeros = jnp.zeros((ROW_SPAN, D), jnp.float32)
    partial = _seg_sum_sc(x, idx2d, zeros)
    return _combine(partial)
